# baseline (speedup 1.0000x reference)
import jax
import jax.numpy as jnp
from jax import lax
from jax.experimental import pallas as pl
from jax.experimental.pallas import tpu as pltpu
from jax.experimental.pallas import tpu_sc as plsc

NF = 8
D = 64
V = 35
DW = NF * D
CH = 64
RPC = CH * NF
GL = 128
NG = RPC // GL
NWORK = 32


def _body(xoff_hbm, wcat_hbm, out_hbm, idx_v, tbl_sh, st0, st1,
          gsem0, gsem1, wsem0, wsem1):
    nrow = idx_v.shape[0]
    nch = nrow // NG
    wid = lax.axis_index("s") * 2 + lax.axis_index("c")

    @pl.when(lax.axis_index("s") == 0)
    def _():
        pltpu.sync_copy(wcat_hbm, tbl_sh)
    pltpu.sync_copy(xoff_hbm.at[wid], idx_v)
    plsc.subcore_barrier()

    stages = (st0, st1)
    gsems = (gsem0, gsem1)
    wsems = (wsem0, wsem1)

    def step(c, slot):
        stage, gsem, wsem = stages[slot], gsems[slot], wsems[slot]

        @pl.when(c >= 2)
        def _():
            pltpu.make_async_copy(
                stage, out_hbm.at[wid, pl.ds(0, RPC)], wsem).wait()

        for q in range(NG):
            pltpu.async_copy(
                tbl_sh.at[idx_v.at[c * NG + q]],
                stage.at[pl.ds(q * GL, GL)], gsem)
        for q in range(NG):
            pltpu.make_async_copy(
                tbl_sh.at[idx_v.at[0]],
                stage.at[pl.ds(q * GL, GL)], gsem).wait()

        pltpu.async_copy(
            stage, out_hbm.at[wid, pl.ds(c * RPC, RPC)], wsem)

    def pair(o, _):
        for phase in range(2):
            step(2 * o + phase, phase)
        return 0
    lax.fori_loop(0, nch // 2, pair, 0)

    for slot in range(2):
        pltpu.make_async_copy(
            stages[slot], out_hbm.at[wid, pl.ds(0, RPC)], wsems[slot]).wait()


def kernel(x, W0, W1, W2, W3, W4, W5, W6, W7):
    B, nf, L = x.shape
    assert nf == NF and B == NWORK and (L * NF) % (2 * NG * GL) == 0
    tables = (W0, W1, W2, W3, W4, W5, W6, W7)
    wcat = jnp.concatenate([w[:V] for w in tables], axis=0)
    xoff = x.astype(jnp.int32) + (V * jnp.arange(NF, dtype=jnp.int32))[None, :, None]
    xoff = xoff.transpose(0, 2, 1).reshape(B, (L * NF) // GL, GL)

    mesh = plsc.VectorSubcoreMesh(core_axis_name="c", subcore_axis_name="s")
    f = pl.kernel(
        _body,
        compiler_params=pltpu.CompilerParams(
            use_tc_tiling_on_sc=False, needs_layout_passes=False),
        out_type=jax.ShapeDtypeStruct((B, L * NF, D), jnp.float32),
        mesh=mesh,
        scratch_types=[
            pltpu.VMEM(((L * NF) // GL, GL), jnp.int32),
            pltpu.VMEM_SHARED((NF * V, D), jnp.float32),
            pltpu.VMEM((RPC, D), jnp.float32),
            pltpu.VMEM((RPC, D), jnp.float32),
            pltpu.SemaphoreType.DMA,
            pltpu.SemaphoreType.DMA,
            pltpu.SemaphoreType.DMA,
            pltpu.SemaphoreType.DMA,
        ],
    )
    out = f(xoff, wcat)
    return out.reshape(B, L, DW)
